# baseline (speedup 1.0000x reference)
import jax
import jax.numpy as jnp
from jax import lax
from jax.experimental import pallas as pl
from jax.experimental.pallas import tpu as pltpu
from jax.experimental.pallas import tpu_sc as plsc

E = 512
M = 16
S = 8
H = 768
NW = 32
E_PER_W = E // NW
IDX_PER_W = E_PER_W * M * S
NBUF = 4
M_PER_Q = M // NBUF
ROWS_Q = M_PER_Q * S
NCHUNK = E_PER_W * NBUF
LANES = 16


def _tree_sum(vals):
    while len(vals) > 1:
        vals = [vals[i] + vals[i + 1] for i in range(0, len(vals) - 1, 2)] + (
            [vals[-1]] if len(vals) % 2 else []
        )
    return vals[0]


def _sc_body(idx_hbm, tok_hbm, men_out, ent_out, idx_v, gbufs, mstg, ents, sems):
    sem_g = sems[0:NBUF]
    sem_m = sems[NBUF:2 * NBUF]
    sem_e = sems[2 * NBUF:]
    c = lax.axis_index("c")
    s = lax.axis_index("s")
    wid = s * 2 + c
    base_e = wid * E_PER_W

    pltpu.sync_copy(idx_hbm.at[pl.ds(wid * IDX_PER_W, IDX_PER_W)], idx_v)

    def idx_slice(k):
        return idx_v.at[pl.ds(k * ROWS_Q, ROWS_Q)]

    def start_gather(k, b):
        pltpu.async_copy(tok_hbm.at[idx_slice(k)], gbufs[b], sem_g[b])

    def wait_gather(k, b):
        pltpu.make_async_copy(tok_hbm.at[idx_slice(k)], gbufs[b], sem_g[b]).wait()

    def compute_quarter(buf, men, ent, q):
        @pl.loop(0, H, step=LANES)
        def _chunk(cc):
            accs = []
            for m in range(M_PER_Q):
                r0 = m * S
                acc = _tree_sum(
                    [buf[r0 + r, pl.ds(cc, LANES)] for r in range(S)]
                )
                men[m, pl.ds(cc, LANES)] = acc * (1.0 / S)
                accs.append(acc)
            eacc = _tree_sum(accs) * (1.0 / (M * S))
            if q == 0:
                ent[0, pl.ds(cc, LANES)] = eacc
            else:
                plsc.addupdate(ent.at[0, pl.ds(cc, LANES)], eacc)

    for b in range(NBUF - 1):
        start_gather(b, b)

    @pl.loop(0, NCHUNK, step=2 * NBUF)
    def _entity_pair(i):
        for ei in range(2):
            ent = ents[ei]
            for q in range(NBUF):
                b = (ei * NBUF + q) % NBUF
                k = i + ei * NBUF + q
                e = base_e + lax.div(k, NBUF)

                wait_gather(k, b)

                k_pre = k + (NBUF - 1)

                @pl.when(k_pre < NCHUNK)
                def _prefetch():
                    start_gather(k_pre, (b + NBUF - 1) % NBUF)

                @pl.when(k >= NBUF)
                def _drain_m():
                    pltpu.make_async_copy(
                        mstg[b],
                        men_out.at[pl.ds(e * M + q * M_PER_Q, M_PER_Q)],
                        sem_m[b],
                    ).wait()

                if q == 0:

                    @pl.when(k >= 2 * NBUF)
                    def _drain_e():
                        pltpu.make_async_copy(
                            ent, ent_out.at[pl.ds(e, 1)], sem_e[ei]
                        ).wait()

                compute_quarter(gbufs[b], mstg[b], ent, q)

                pltpu.async_copy(
                    mstg[b],
                    men_out.at[pl.ds(e * M + q * M_PER_Q, M_PER_Q)],
                    sem_m[b],
                )
                if q == NBUF - 1:
                    pltpu.async_copy(ent, ent_out.at[pl.ds(e, 1)], sem_e[ei])

    for b in range(NBUF):
        pltpu.make_async_copy(
            mstg[b], men_out.at[pl.ds(0, M_PER_Q)], sem_m[b]
        ).wait()
    for ei in range(2):
        pltpu.make_async_copy(ents[ei], ent_out.at[pl.ds(0, 1)], sem_e[ei]).wait()


@jax.jit
def _sc_entity_repr(token_repr, idx_flat):
    mesh = plsc.VectorSubcoreMesh(core_axis_name="c", subcore_axis_name="s")
    run = pl.kernel(
        _sc_body,
        out_type=[
            jax.ShapeDtypeStruct((E * M, H), jnp.float32),
            jax.ShapeDtypeStruct((E, H), jnp.float32),
        ],
        mesh=mesh,
        scratch_types=[
            pltpu.VMEM((IDX_PER_W,), jnp.int32),
            [pltpu.VMEM((ROWS_Q, H), jnp.float32) for _ in range(NBUF)],
            [pltpu.VMEM((M_PER_Q, H), jnp.float32) for _ in range(NBUF)],
            [pltpu.VMEM((1, H), jnp.float32) for _ in range(2)],
            [pltpu.SemaphoreType.DMA for _ in range(2 * NBUF + 2)],
        ],
    )
    return run(idx_flat, token_repr)


def kernel(token_repr, pos_idx):
    idx_flat = pos_idx.astype(jnp.int32).reshape(-1)
    men, ent = _sc_entity_repr(token_repr, idx_flat)
    mentions_reprs = men.reshape(E, M, H)
    mentions_mask = jnp.ones((E, M), dtype=jnp.float32)
    return (ent, mentions_reprs, mentions_mask)


# speedup vs baseline: 1.6655x; 1.6655x over previous
import jax
import jax.numpy as jnp
from jax import lax
from jax.experimental import pallas as pl
from jax.experimental.pallas import tpu as pltpu
from jax.experimental.pallas import tpu_sc as plsc

E = 512
M = 16
S = 8
H = 768
NW = 32
E_PER_W = E // NW
IDX_PER_W = E_PER_W * M * S
ROWS_HALF = M * S // 2
LANES = 16


def _tree_sum(vals):
    while len(vals) > 1:
        vals = [vals[i] + vals[i + 1] for i in range(0, len(vals) - 1, 2)] + (
            [vals[-1]] if len(vals) % 2 else []
        )
    return vals[0]


def _sc_body(idx_hbm, tok_hbm, men_out, ent_out, idx_v, gbufs, mens, ents, sems):
    semA, semB, semW0, semW1 = sems
    sem_w = (semW0, semW1)
    c = lax.axis_index("c")
    s = lax.axis_index("s")
    wid = s * 2 + c
    base_e = wid * E_PER_W

    pltpu.sync_copy(idx_hbm.at[pl.ds(wid * IDX_PER_W, IDX_PER_W)], idx_v)

    def idx_slice(local_e, half):
        return idx_v.at[pl.ds(local_e * (M * S) + half * ROWS_HALF, ROWS_HALF)]

    def start_gather(local_e, half, buf, sem):
        pltpu.async_copy(tok_hbm.at[idx_slice(local_e, half)], buf, sem)

    def wait_gather(local_e, half, buf, sem):
        pltpu.make_async_copy(tok_hbm.at[idx_slice(local_e, half)], buf, sem).wait()

    def compute_half(buf, men, ent, half):
        @pl.loop(0, H, step=LANES)
        def _chunk(cc):
            accs = []
            for m in range(M // 2):
                r0 = m * S
                acc = _tree_sum(
                    [buf[r0 + r, pl.ds(cc, LANES)] for r in range(S)]
                )
                men[half * (M // 2) + m, pl.ds(cc, LANES)] = acc * (1.0 / S)
                accs.append(acc)
            eacc = _tree_sum(accs) * (1.0 / (M * S))
            if half == 0:
                ent[0, pl.ds(cc, LANES)] = eacc
            else:
                plsc.addupdate(ent.at[0, pl.ds(cc, LANES)], eacc)

    start_gather(0, 0, gbufs[0], semA)

    @pl.loop(0, E_PER_W, step=2)
    def _entity_pair(i):
        for ei in range(2):
            local_e = i + ei
            e = base_e + local_e
            men = mens[ei]
            ent = ents[ei]

            start_gather(local_e, 1, gbufs[1], semB)
            wait_gather(local_e, 0, gbufs[0], semA)

            @pl.when(local_e >= 2)
            def _drain():
                pltpu.make_async_copy(
                    men, men_out.at[pl.ds(e * M, M)], sem_w[ei]
                ).wait()
                pltpu.make_async_copy(
                    ent, ent_out.at[pl.ds(e, 1)], sem_w[ei]
                ).wait()

            compute_half(gbufs[0], men, ent, 0)

            start_gather((local_e + 1) & (E_PER_W - 1), 0, gbufs[0], semA)
            wait_gather(local_e, 1, gbufs[1], semB)
            compute_half(gbufs[1], men, ent, 1)

            pltpu.async_copy(men, men_out.at[pl.ds(e * M, M)], sem_w[ei])
            pltpu.async_copy(ent, ent_out.at[pl.ds(e, 1)], sem_w[ei])

    for ei in range(2):
        pltpu.make_async_copy(
            mens[ei], men_out.at[pl.ds(0, M)], sem_w[ei]
        ).wait()
        pltpu.make_async_copy(ents[ei], ent_out.at[pl.ds(0, 1)], sem_w[ei]).wait()


@jax.jit
def _sc_entity_repr(token_repr, idx_flat):
    mesh = plsc.VectorSubcoreMesh(core_axis_name="c", subcore_axis_name="s")
    run = pl.kernel(
        _sc_body,
        out_type=[
            jax.ShapeDtypeStruct((E * M, H), jnp.float32),
            jax.ShapeDtypeStruct((E, H), jnp.float32),
        ],
        mesh=mesh,
        scratch_types=[
            pltpu.VMEM((IDX_PER_W,), jnp.int32),
            [pltpu.VMEM((ROWS_HALF, H), jnp.float32) for _ in range(2)],
            [pltpu.VMEM((M, H), jnp.float32) for _ in range(2)],
            [pltpu.VMEM((1, H), jnp.float32) for _ in range(2)],
            [pltpu.SemaphoreType.DMA for _ in range(4)],
        ],
    )
    return run(idx_flat, token_repr)


def kernel(token_repr, pos_idx):
    idx_flat = pos_idx.astype(jnp.int32).reshape(-1)
    men, ent = _sc_entity_repr(token_repr, idx_flat)
    mentions_reprs = men.reshape(E, M, H)
    mentions_mask = jnp.ones((E, M), dtype=jnp.float32)
    return (ent, mentions_reprs, mentions_mask)
